# baseline (speedup 1.0000x reference)
import jax
import jax.numpy as jnp
from jax import lax
from jax.experimental import pallas as pl
from jax.experimental.pallas import tpu as pltpu
from jax.experimental.pallas import tpu_sc as plsc

_N = 4096
_NV = _N // 16
_EPS = 1e-8
_BIG = _N
_MININT = -2147483648
_UNROLL = 8


def _iota16():
    return lax.iota(jnp.int32, 16)


def _keys_from_raw(x):
    x = jnp.where(x == 0.0, 0.0, x)
    i = lax.bitcast_convert_type(x, jnp.int32)
    return jnp.where(i < 0, ~i, i | jnp.int32(_MININT))


def _radix_pass(src_k, src_v, dst_k, dst_v, cnt, tots, shift, twist_out,
                first, raw=None):
    ones = jnp.ones((16,), jnp.int32)
    lane0 = _iota16() == 0

    def digits(k):
        d = jnp.bitwise_and(lax.shift_right_logical(k, shift), 255)
        return (d << 4) + _iota16()

    def s1(b, c):
        off = b * 16
        if raw is not None:
            k = _keys_from_raw(raw[pl.ds(off, 16)])
            src_k[pl.ds(off, 16)] = k
        else:
            k = src_k[pl.ds(off, 16)]
        plsc.addupdate_scatter(cnt, [digits(k)], ones)
        return c

    lax.fori_loop(0, _NV, s1, 0, unroll=_UNROLL)

    def csum_a(dg, c):
        c0 = cnt[pl.ds(dg * 16, 16)]
        incl = plsc.cumsum(c0)
        cnt[pl.ds(dg * 16, 16)] = incl - c0
        tot = lax.reduce_max(incl, (0,))
        plsc.store_scatter(
            tots, [jnp.full((16,), dg, jnp.int32)],
            jnp.full((16,), tot), mask=lane0,
        )
        return c

    lax.fori_loop(0, _NV, csum_a, 0, unroll=_UNROLL)

    def csum_b(i, carry):
        t = tots[pl.ds(i * 16, 16)]
        incl = plsc.cumsum(t)
        tots[pl.ds(i * 16, 16)] = incl - t + carry
        return carry + lax.reduce_max(incl, (0,))

    lax.fori_loop(0, 16, csum_b, jnp.int32(0))

    def s2(b, c):
        off = b * 16
        k = src_k[pl.ds(off, 16)]
        d = jnp.bitwise_and(lax.shift_right_logical(k, shift), 255)
        idx = (d << 4) + _iota16()
        pos = (
            plsc.load_gather(cnt, [idx])
            + plsc.load_gather(tots, [d])
        )
        v = _iota16() + off if first else src_v[pl.ds(off, 16)]
        if twist_out:
            w = (jnp.bitwise_and(pos, 255) << 4) | lax.shift_right_logical(
                pos, 8
            )
        else:
            w = pos
        plsc.store_scatter(dst_k, [w], k)
        plsc.store_scatter(dst_v, [w], v)
        plsc.addupdate_scatter(cnt, [idx], ones)
        return c

    lax.fori_loop(0, _NV, s2, 0, unroll=_UNROLL)


def _rank_scatter(kf, vf, st, rdst):
    def fwd(b, carry):
        off = b * 16
        k = kf[pl.ds(off, 16)]
        pidx = _iota16() + off
        prevk = plsc.load_gather(kf, [jnp.maximum(pidx - 1, 0)])
        bnd = jnp.logical_or(k != prevk, pidx == 0)
        cand = jnp.where(bnd, pidx, 0)
        cm = jnp.maximum(plsc.cummax(cand), carry)
        st[pl.ds(off, 16)] = cm
        return lax.reduce_max(cm, (0,))

    lax.fori_loop(0, _NV, fwd, jnp.int32(0), unroll=_UNROLL)

    def bwd(t, carry):
        b = _NV - 1 - t
        off = b * 16
        k = kf[pl.ds(off, 16)]
        pidx = _iota16() + off
        nxtk = plsc.load_gather(kf, [jnp.minimum(pidx + 1, _N - 1)])
        endb = jnp.logical_or(k != nxtk, pidx == _N - 1)
        cand = jnp.where(endb, pidx, _BIG)
        sfx = lax.rev(-plsc.cummax(-lax.rev(cand, (0,))), (0,))
        end = jnp.minimum(sfx, carry)
        s = st[pl.ds(off, 16)]
        rank_c = (s + end).astype(jnp.float32) * 0.5 + (1.0 - (_N + 1) / 2.0)
        v = vf[pl.ds(off, 16)]
        plsc.store_scatter(rdst, [v], rank_c)
        return lax.reduce_min(end, (0,))

    lax.fori_loop(0, _NV, bwd, jnp.int32(_N), unroll=_UNROLL)


def kernel(pred_y, true_y):
    b, n = pred_y.shape
    mesh = plsc.VectorSubcoreMesh(core_axis_name="c", subcore_axis_name="s")
    nworkers = mesh.num_cores * mesh.num_subcores
    rows_per = b // nworkers

    def body(x_hbm, y_hbm, out_hbm, raw, kA, kB, vA, vB,
             c0, c1, c2, c3, tots, st, rx, ry, res):
        wid = lax.axis_index("s") * mesh.num_cores + lax.axis_index("c")
        zeros = jnp.zeros((16,), jnp.int32)

        def do_array(src_hbm, r, rdst):
            pltpu.sync_copy(src_hbm.at[r], raw)

            def zero(i, c):
                c0[pl.ds(i * 16, 16)] = zeros
                c1[pl.ds(i * 16, 16)] = zeros
                c2[pl.ds(i * 16, 16)] = zeros
                c3[pl.ds(i * 16, 16)] = zeros
                return c

            lax.fori_loop(0, _NV, zero, 0, unroll=_UNROLL)
            _radix_pass(kA, vA, kB, vB, c0, tots, 0, True, True, raw=raw)
            _radix_pass(kB, vB, kA, vA, c1, tots, 8, True, False)
            _radix_pass(kA, vA, kB, vB, c2, tots, 16, True, False)
            _radix_pass(kB, vB, kA, vA, c3, tots, 24, False, False)
            _rank_scatter(kA, vA, st, rdst)

        def row_body(rloc, carry):
            r = wid * rows_per + rloc
            do_array(x_hbm, r, rx)
            do_array(y_hbm, r, ry)

            def dot_body(i, c):
                axy, axx, ayy = c
                off = i * 16
                a = rx[pl.ds(off, 16)]
                cc = ry[pl.ds(off, 16)]
                return axy + a * cc, axx + a * a, ayy + cc * cc

            z = jnp.zeros((16,), jnp.float32)
            axy, axx, ayy = lax.fori_loop(
                0, _NV, dot_body, (z, z, z), unroll=_UNROLL
            )
            num = lax.reduce_sum(axy, (0,))
            den2 = lax.reduce_sum(axx, (0,)) * lax.reduce_sum(ayy, (0,))
            idx_n = jnp.full((16,), rloc, jnp.int32)
            idx_d = jnp.full((16,), rloc + 8, jnp.int32)
            lane0 = _iota16() == 0
            plsc.store_scatter(res, [idx_n], jnp.full((16,), num), mask=lane0)
            plsc.store_scatter(res, [idx_d], jnp.full((16,), den2), mask=lane0)
            return carry

        lax.fori_loop(0, rows_per, row_body, 0)
        pltpu.sync_copy(res, out_hbm.at[wid])

    k = pl.kernel(
        body,
        out_type=jax.ShapeDtypeStruct((nworkers, 16), jnp.float32),
        mesh=mesh,
        compiler_params=pltpu.CompilerParams(needs_layout_passes=False),
        scratch_types=[
            pltpu.VMEM((_N,), jnp.float32),
            pltpu.VMEM((_N,), jnp.int32),
            pltpu.VMEM((_N,), jnp.int32),
            pltpu.VMEM((_N,), jnp.int32),
            pltpu.VMEM((_N,), jnp.int32),
            pltpu.VMEM((_N,), jnp.int32),
            pltpu.VMEM((_N,), jnp.int32),
            pltpu.VMEM((_N,), jnp.int32),
            pltpu.VMEM((_N,), jnp.int32),
            pltpu.VMEM((256,), jnp.int32),
            pltpu.VMEM((_N,), jnp.int32),
            pltpu.VMEM((_N,), jnp.float32),
            pltpu.VMEM((_N,), jnp.float32),
            pltpu.VMEM((16,), jnp.float32),
        ],
    )
    out = k(pred_y, true_y)
    num = out[:, 0:8].reshape(b)
    den2 = out[:, 8:16].reshape(b)
    return num / jnp.sqrt(den2 + _EPS)


# speedup vs baseline: 1.4835x; 1.4835x over previous
import jax
import jax.numpy as jnp
from jax import lax
from jax.experimental import pallas as pl
from jax.experimental.pallas import tpu as pltpu
from jax.experimental.pallas import tpu_sc as plsc

_N = 4096
_NV = _N // 16
_EPS = 1e-8
_BIG = _N
_MININT = -2147483648
_UNROLL = 8


def _iota16():
    return lax.iota(jnp.int32, 16)


def _keys_from_raw(x):
    x = jnp.where(x == 0.0, 0.0, x)
    i = lax.bitcast_convert_type(x, jnp.int32)
    return jnp.where(i < 0, ~i, i | jnp.int32(_MININT))


def _radix_pass(src_k, src_v, dst_k, dst_v, cnt, occ, shift, twist_out,
                first, raw=None):
    ones = jnp.ones((16,), jnp.int32)

    def digits(k):
        d = jnp.bitwise_and(lax.shift_right_logical(k, shift), 255)
        return (d << 4) + _iota16()

    def s1(b, c):
        off = b * 16
        if raw is not None:
            k = _keys_from_raw(raw[pl.ds(off, 16)])
            src_k[pl.ds(off, 16)] = k
        else:
            k = src_k[pl.ds(off, 16)]
        idx = digits(k)
        occ[pl.ds(off, 16)] = plsc.load_gather(cnt, [idx])
        plsc.addupdate_scatter(cnt, [idx], ones)
        return c

    lax.fori_loop(0, _NV, s1, 0, unroll=_UNROLL)

    def csum(dg, carry):
        c0 = cnt[pl.ds(dg * 16, 16)]
        incl = plsc.cumsum(c0)
        tot = lax.reduce_sum(c0, (0,))
        cnt[pl.ds(dg * 16, 16)] = incl - c0 + carry
        return carry + tot

    lax.fori_loop(0, _NV, csum, jnp.int32(0), unroll=_UNROLL)

    def s2(b, c):
        off = b * 16
        k = src_k[pl.ds(off, 16)]
        idx = digits(k)
        pos = plsc.load_gather(cnt, [idx]) + occ[pl.ds(off, 16)]
        v = _iota16() + off if first else src_v[pl.ds(off, 16)]
        if twist_out:
            w = (jnp.bitwise_and(pos, 255) << 4) | lax.shift_right_logical(
                pos, 8
            )
        else:
            w = pos
        plsc.store_scatter(dst_k, [w], k)
        plsc.store_scatter(dst_v, [w], v)
        return c

    lax.fori_loop(0, _NV, s2, 0, unroll=_UNROLL)


def _rank_scatter(kf, vf, st, rdst):
    def fwd(b, carry):
        off = b * 16
        k = kf[pl.ds(off, 16)]
        pidx = _iota16() + off
        prevk = plsc.load_gather(kf, [jnp.maximum(pidx - 1, 0)])
        bnd = jnp.logical_or(k != prevk, pidx == 0)
        cand = jnp.where(bnd, pidx, 0)
        cm = jnp.maximum(plsc.cummax(cand), carry)
        st[pl.ds(off, 16)] = cm
        return lax.reduce_max(cm, (0,))

    lax.fori_loop(0, _NV, fwd, jnp.int32(0), unroll=_UNROLL)

    def bwd(t, carry):
        b = _NV - 1 - t
        off = b * 16
        k = kf[pl.ds(off, 16)]
        pidx = _iota16() + off
        nxtk = plsc.load_gather(kf, [jnp.minimum(pidx + 1, _N - 1)])
        endb = jnp.logical_or(k != nxtk, pidx == _N - 1)
        cand = jnp.where(endb, pidx, _BIG)
        sfx = lax.rev(-plsc.cummax(-lax.rev(cand, (0,))), (0,))
        end = jnp.minimum(sfx, carry)
        s = st[pl.ds(off, 16)]
        rank_c = (s + end).astype(jnp.float32) * 0.5 + (1.0 - (_N + 1) / 2.0)
        v = vf[pl.ds(off, 16)]
        plsc.store_scatter(rdst, [v], rank_c)
        return lax.reduce_min(end, (0,))

    lax.fori_loop(0, _NV, bwd, jnp.int32(_N), unroll=_UNROLL)


def kernel(pred_y, true_y):
    b, n = pred_y.shape
    mesh = plsc.VectorSubcoreMesh(core_axis_name="c", subcore_axis_name="s")
    nworkers = mesh.num_cores * mesh.num_subcores
    rows_per = b // nworkers

    def body(x_hbm, y_hbm, out_hbm, raw, kA, kB, vA, vB,
             c0, c1, c2, c3, st, rx, ry, res):
        wid = lax.axis_index("s") * mesh.num_cores + lax.axis_index("c")
        zeros = jnp.zeros((16,), jnp.int32)

        def do_array(src_hbm, r, rdst):
            pltpu.sync_copy(src_hbm.at[r], raw)

            def zero(i, c):
                c0[pl.ds(i * 16, 16)] = zeros
                c1[pl.ds(i * 16, 16)] = zeros
                c2[pl.ds(i * 16, 16)] = zeros
                c3[pl.ds(i * 16, 16)] = zeros
                return c

            lax.fori_loop(0, _NV, zero, 0, unroll=_UNROLL)
            _radix_pass(kA, vA, kB, vB, c0, st, 0, True, True, raw=raw)
            _radix_pass(kB, vB, kA, vA, c1, st, 8, True, False)
            _radix_pass(kA, vA, kB, vB, c2, st, 16, True, False)
            _radix_pass(kB, vB, kA, vA, c3, st, 24, False, False)
            _rank_scatter(kA, vA, st, rdst)

        def row_body(rloc, carry):
            r = wid * rows_per + rloc
            do_array(x_hbm, r, rx)
            do_array(y_hbm, r, ry)

            def dot_body(i, c):
                axy, axx, ayy = c
                off = i * 16
                a = rx[pl.ds(off, 16)]
                cc = ry[pl.ds(off, 16)]
                return axy + a * cc, axx + a * a, ayy + cc * cc

            z = jnp.zeros((16,), jnp.float32)
            axy, axx, ayy = lax.fori_loop(
                0, _NV, dot_body, (z, z, z), unroll=_UNROLL
            )
            num = lax.reduce_sum(axy, (0,))
            den2 = lax.reduce_sum(axx, (0,)) * lax.reduce_sum(ayy, (0,))
            idx_n = jnp.full((16,), rloc, jnp.int32)
            idx_d = jnp.full((16,), rloc + 8, jnp.int32)
            lane0 = _iota16() == 0
            plsc.store_scatter(res, [idx_n], jnp.full((16,), num), mask=lane0)
            plsc.store_scatter(res, [idx_d], jnp.full((16,), den2), mask=lane0)
            return carry

        lax.fori_loop(0, rows_per, row_body, 0)
        pltpu.sync_copy(res, out_hbm.at[wid])

    k = pl.kernel(
        body,
        out_type=jax.ShapeDtypeStruct((nworkers, 16), jnp.float32),
        mesh=mesh,
        compiler_params=pltpu.CompilerParams(needs_layout_passes=False),
        scratch_types=[
            pltpu.VMEM((_N,), jnp.float32),
            pltpu.VMEM((_N,), jnp.int32),
            pltpu.VMEM((_N,), jnp.int32),
            pltpu.VMEM((_N,), jnp.int32),
            pltpu.VMEM((_N,), jnp.int32),
            pltpu.VMEM((_N,), jnp.int32),
            pltpu.VMEM((_N,), jnp.int32),
            pltpu.VMEM((_N,), jnp.int32),
            pltpu.VMEM((_N,), jnp.int32),
            pltpu.VMEM((_N,), jnp.int32),
            pltpu.VMEM((_N,), jnp.float32),
            pltpu.VMEM((_N,), jnp.float32),
            pltpu.VMEM((16,), jnp.float32),
        ],
    )
    out = k(pred_y, true_y)
    num = out[:, 0:8].reshape(b)
    den2 = out[:, 8:16].reshape(b)
    return num / jnp.sqrt(den2 + _EPS)


# speedup vs baseline: 1.6502x; 1.1123x over previous
import jax
import jax.numpy as jnp
from jax import lax
from jax.experimental import pallas as pl
from jax.experimental.pallas import tpu as pltpu
from jax.experimental.pallas import tpu_sc as plsc

_N = 4096
_NV = _N // 16
_EPS = 1e-8
_BIG = _N
_MININT = -2147483648
_UNROLL = 16


def _iota16():
    return lax.iota(jnp.int32, 16)


def _keys_from_raw(x):
    x = jnp.where(x == 0.0, 0.0, x)
    i = lax.bitcast_convert_type(x, jnp.int32)
    return jnp.where(i < 0, ~i, i | jnp.int32(_MININT))


def _radix_pass(src_k, src_v, dst_k, dst_v, cnt, shift, twist_out, first,
                raw=None):
    ones = jnp.ones((16,), jnp.int32)

    def digits(k):
        d = jnp.bitwise_and(lax.shift_right_logical(k, shift), 255)
        return (d << 4) + _iota16()

    def s1(b, c):
        off = b * 16
        if raw is not None:
            k = _keys_from_raw(raw[pl.ds(off, 16)])
            src_k[pl.ds(off, 16)] = k
        else:
            k = src_k[pl.ds(off, 16)]
        plsc.addupdate_scatter(cnt, [digits(k)], ones)
        return c

    lax.fori_loop(0, _NV, s1, 0, unroll=_UNROLL)

    def csum(dg, carry):
        c0 = cnt[pl.ds(dg * 16, 16)]
        incl = plsc.cumsum(c0)
        tot = lax.reduce_sum(c0, (0,))
        cnt[pl.ds(dg * 16, 16)] = incl - c0 + carry
        return carry + tot

    lax.fori_loop(0, _NV, csum, jnp.int32(0), unroll=_UNROLL)

    def s2(b, c):
        off = b * 16
        k = src_k[pl.ds(off, 16)]
        idx = digits(k)
        pos = plsc.load_gather(cnt, [idx])
        v = _iota16() + off if first else src_v[pl.ds(off, 16)]
        if twist_out:
            w = (jnp.bitwise_and(pos, 255) << 4) | lax.shift_right_logical(
                pos, 8
            )
        else:
            w = pos
        plsc.store_scatter(dst_k, [w], k)
        plsc.store_scatter(dst_v, [w], v)
        plsc.addupdate_scatter(cnt, [idx], ones)
        return c

    lax.fori_loop(0, _NV, s2, 0, unroll=_UNROLL)


def _rank_scatter(kf, vf, st, rdst):
    def fwd(b, carry):
        off = b * 16
        k = kf[pl.ds(off, 16)]
        pidx = _iota16() + off
        prevk = plsc.load_gather(kf, [jnp.maximum(pidx - 1, 0)])
        bnd = jnp.logical_or(k != prevk, pidx == 0)
        cand = jnp.where(bnd, pidx, 0)
        cm = jnp.maximum(plsc.cummax(cand), carry)
        st[pl.ds(off, 16)] = cm
        return lax.reduce_max(cm, (0,))

    lax.fori_loop(0, _NV, fwd, jnp.int32(0), unroll=_UNROLL)

    def bwd(t, carry):
        b = _NV - 1 - t
        off = b * 16
        k = kf[pl.ds(off, 16)]
        pidx = _iota16() + off
        nxtk = plsc.load_gather(kf, [jnp.minimum(pidx + 1, _N - 1)])
        endb = jnp.logical_or(k != nxtk, pidx == _N - 1)
        cand = jnp.where(endb, pidx, _BIG)
        sfx = lax.rev(-plsc.cummax(-lax.rev(cand, (0,))), (0,))
        end = jnp.minimum(sfx, carry)
        s = st[pl.ds(off, 16)]
        rank_c = (s + end).astype(jnp.float32) * 0.5 + (1.0 - (_N + 1) / 2.0)
        v = vf[pl.ds(off, 16)]
        plsc.store_scatter(rdst, [v], rank_c)
        return lax.reduce_min(end, (0,))

    lax.fori_loop(0, _NV, bwd, jnp.int32(_N), unroll=_UNROLL)


def kernel(pred_y, true_y):
    b, n = pred_y.shape
    mesh = plsc.VectorSubcoreMesh(core_axis_name="c", subcore_axis_name="s")
    nworkers = mesh.num_cores * mesh.num_subcores
    rows_per = b // nworkers

    def body(x_hbm, y_hbm, out_hbm, raw, kA, kB, vA, vB,
             c0, c1, c2, c3, st, rx, ry, res):
        wid = lax.axis_index("s") * mesh.num_cores + lax.axis_index("c")
        zeros = jnp.zeros((16,), jnp.int32)

        def do_array(src_hbm, r, rdst):
            pltpu.sync_copy(src_hbm.at[r], raw)

            def zero(i, c):
                c0[pl.ds(i * 16, 16)] = zeros
                c1[pl.ds(i * 16, 16)] = zeros
                c2[pl.ds(i * 16, 16)] = zeros
                c3[pl.ds(i * 16, 16)] = zeros
                return c

            lax.fori_loop(0, _NV, zero, 0, unroll=_UNROLL)
            _radix_pass(kA, vA, kB, vB, c0, 0, True, True, raw=raw)
            _radix_pass(kB, vB, kA, vA, c1, 8, True, False)
            _radix_pass(kA, vA, kB, vB, c2, 16, True, False)
            _radix_pass(kB, vB, kA, vA, c3, 24, False, False)
            _rank_scatter(kA, vA, st, rdst)

        def row_body(rloc, carry):
            r = wid * rows_per + rloc
            do_array(x_hbm, r, rx)
            do_array(y_hbm, r, ry)

            def dot_body(i, c):
                axy, axx, ayy = c
                off = i * 16
                a = rx[pl.ds(off, 16)]
                cc = ry[pl.ds(off, 16)]
                return axy + a * cc, axx + a * a, ayy + cc * cc

            z = jnp.zeros((16,), jnp.float32)
            axy, axx, ayy = lax.fori_loop(
                0, _NV, dot_body, (z, z, z), unroll=_UNROLL
            )
            num = lax.reduce_sum(axy, (0,))
            den2 = lax.reduce_sum(axx, (0,)) * lax.reduce_sum(ayy, (0,))
            idx_n = jnp.full((16,), rloc, jnp.int32)
            idx_d = jnp.full((16,), rloc + 8, jnp.int32)
            lane0 = _iota16() == 0
            plsc.store_scatter(res, [idx_n], jnp.full((16,), num), mask=lane0)
            plsc.store_scatter(res, [idx_d], jnp.full((16,), den2), mask=lane0)
            return carry

        lax.fori_loop(0, rows_per, row_body, 0)
        pltpu.sync_copy(res, out_hbm.at[wid])

    k = pl.kernel(
        body,
        out_type=jax.ShapeDtypeStruct((nworkers, 16), jnp.float32),
        mesh=mesh,
        compiler_params=pltpu.CompilerParams(needs_layout_passes=False),
        scratch_types=[
            pltpu.VMEM((_N,), jnp.float32),
            pltpu.VMEM((_N,), jnp.int32),
            pltpu.VMEM((_N,), jnp.int32),
            pltpu.VMEM((_N,), jnp.int32),
            pltpu.VMEM((_N,), jnp.int32),
            pltpu.VMEM((_N,), jnp.int32),
            pltpu.VMEM((_N,), jnp.int32),
            pltpu.VMEM((_N,), jnp.int32),
            pltpu.VMEM((_N,), jnp.int32),
            pltpu.VMEM((_N,), jnp.int32),
            pltpu.VMEM((_N,), jnp.float32),
            pltpu.VMEM((_N,), jnp.float32),
            pltpu.VMEM((16,), jnp.float32),
        ],
    )
    out = k(pred_y, true_y)
    num = out[:, 0:8].reshape(b)
    den2 = out[:, 8:16].reshape(b)
    return num / jnp.sqrt(den2 + _EPS)


# speedup vs baseline: 1.7058x; 1.0337x over previous
import jax
import jax.numpy as jnp
from jax import lax
from jax.experimental import pallas as pl
from jax.experimental.pallas import tpu as pltpu
from jax.experimental.pallas import tpu_sc as plsc

_N = 4096
_NV = _N // 16
_EPS = 1e-8
_BIG = _N
_MININT = -2147483648
_UNROLL = 16


def _iota16():
    return lax.iota(jnp.int32, 16)


def _keys_from_raw(x):
    x = jnp.where(x == 0.0, 0.0, x)
    i = lax.bitcast_convert_type(x, jnp.int32)
    return jnp.where(i < 0, ~i, i | jnp.int32(_MININT))


def _radix_pass(src_k, src_v, dst_k, dst_v, cnt, shift, twist_out, first,
                raw=None):
    ones = jnp.ones((16,), jnp.int32)

    def digits(k):
        d = jnp.bitwise_and(lax.shift_right_logical(k, shift), 255)
        return (d << 4) + _iota16()

    def s1(b, c):
        off = b * 16
        if raw is not None:
            k = _keys_from_raw(raw[pl.ds(off, 16)])
            src_k[pl.ds(off, 16)] = k
        else:
            k = src_k[pl.ds(off, 16)]
        plsc.addupdate_scatter(cnt, [digits(k)], ones)
        return c

    lax.fori_loop(0, _NV, s1, 0, unroll=_UNROLL)

    def csum(dg, carry):
        c0 = cnt[pl.ds(dg * 16, 16)]
        incl = plsc.cumsum(c0)
        tot = lax.reduce_sum(c0, (0,))
        cnt[pl.ds(dg * 16, 16)] = incl - c0 + carry
        return carry + tot

    lax.fori_loop(0, _NV, csum, jnp.int32(0), unroll=_UNROLL)

    def s2(b, c):
        off = b * 16
        k = src_k[pl.ds(off, 16)]
        idx = digits(k)
        pos = plsc.load_gather(cnt, [idx])
        v = _iota16() + off if first else src_v[pl.ds(off, 16)]
        if twist_out:
            w = (jnp.bitwise_and(pos, 255) << 4) | lax.shift_right_logical(
                pos, 8
            )
        else:
            w = pos
        plsc.store_scatter(dst_k, [w], k)
        plsc.store_scatter(dst_v, [w], v)
        plsc.addupdate_scatter(cnt, [idx], ones)
        return c

    lax.fori_loop(0, _NV, s2, 0, unroll=_UNROLL)


def _rank_scatter(kf, vf, st, rdst):
    def fwd(b, carry):
        off = b * 16
        k = kf[pl.ds(off, 16)]
        pidx = _iota16() + off
        prevk = plsc.load_gather(kf, [jnp.maximum(pidx - 1, 0)])
        bnd = jnp.logical_or(k != prevk, pidx == 0)
        cand = jnp.where(bnd, pidx, 0)
        cm = jnp.maximum(plsc.cummax(cand), carry)
        st[pl.ds(off, 16)] = cm
        return lax.reduce_max(cm, (0,))

    lax.fori_loop(0, _NV, fwd, jnp.int32(0), unroll=_UNROLL)

    def bwd(t, carry):
        ec, axx = carry
        b = _NV - 1 - t
        off = b * 16
        k = kf[pl.ds(off, 16)]
        pidx = _iota16() + off
        nxtk = plsc.load_gather(kf, [jnp.minimum(pidx + 1, _N - 1)])
        endb = jnp.logical_or(k != nxtk, pidx == _N - 1)
        cand = jnp.where(endb, pidx, _BIG)
        sfx = lax.rev(-plsc.cummax(-lax.rev(cand, (0,))), (0,))
        end = jnp.minimum(sfx, ec)
        s = st[pl.ds(off, 16)]
        rank_c = (s + end).astype(jnp.float32) * 0.5 + (1.0 - (_N + 1) / 2.0)
        v = vf[pl.ds(off, 16)]
        plsc.store_scatter(rdst, [v], rank_c)
        return lax.reduce_min(end, (0,)), axx + rank_c * rank_c

    z = jnp.zeros((16,), jnp.float32)
    _, axx = lax.fori_loop(0, _NV, bwd, (jnp.int32(_N), z), unroll=_UNROLL)
    return lax.reduce_sum(axx, (0,))


def _rank_dot(kf, vf, st, rx):
    def fwd(b, carry):
        off = b * 16
        k = kf[pl.ds(off, 16)]
        pidx = _iota16() + off
        prevk = plsc.load_gather(kf, [jnp.maximum(pidx - 1, 0)])
        bnd = jnp.logical_or(k != prevk, pidx == 0)
        cand = jnp.where(bnd, pidx, 0)
        cm = jnp.maximum(plsc.cummax(cand), carry)
        st[pl.ds(off, 16)] = cm
        return lax.reduce_max(cm, (0,))

    lax.fori_loop(0, _NV, fwd, jnp.int32(0), unroll=_UNROLL)

    def bwd(t, carry):
        ec, axy, ayy = carry
        b = _NV - 1 - t
        off = b * 16
        k = kf[pl.ds(off, 16)]
        pidx = _iota16() + off
        nxtk = plsc.load_gather(kf, [jnp.minimum(pidx + 1, _N - 1)])
        endb = jnp.logical_or(k != nxtk, pidx == _N - 1)
        cand = jnp.where(endb, pidx, _BIG)
        sfx = lax.rev(-plsc.cummax(-lax.rev(cand, (0,))), (0,))
        end = jnp.minimum(sfx, ec)
        s = st[pl.ds(off, 16)]
        rank_c = (s + end).astype(jnp.float32) * 0.5 + (1.0 - (_N + 1) / 2.0)
        v = vf[pl.ds(off, 16)]
        rxv = plsc.load_gather(rx, [v])
        return (
            lax.reduce_min(end, (0,)),
            axy + rxv * rank_c,
            ayy + rank_c * rank_c,
        )

    z = jnp.zeros((16,), jnp.float32)
    _, axy, ayy = lax.fori_loop(
        0, _NV, bwd, (jnp.int32(_N), z, z), unroll=_UNROLL
    )
    return lax.reduce_sum(axy, (0,)), lax.reduce_sum(ayy, (0,))


def kernel(pred_y, true_y):
    b, n = pred_y.shape
    mesh = plsc.VectorSubcoreMesh(core_axis_name="c", subcore_axis_name="s")
    nworkers = mesh.num_cores * mesh.num_subcores
    rows_per = b // nworkers

    def body(x_hbm, y_hbm, out_hbm, rawA, rawB, kA, kB, vA, vB,
             c0, c1, c2, c3, st, rx, res, semA, semB):
        wid = lax.axis_index("s") * mesh.num_cores + lax.axis_index("c")
        zeros = jnp.zeros((16,), jnp.int32)

        def sort_array(raw):
            def zero(i, c):
                c0[pl.ds(i * 16, 16)] = zeros
                c1[pl.ds(i * 16, 16)] = zeros
                c2[pl.ds(i * 16, 16)] = zeros
                c3[pl.ds(i * 16, 16)] = zeros
                return c

            lax.fori_loop(0, _NV, zero, 0, unroll=_UNROLL)
            _radix_pass(kA, vA, kB, vB, c0, 0, True, True, raw=raw)
            _radix_pass(kB, vB, kA, vA, c1, 8, True, False)
            _radix_pass(kA, vA, kB, vB, c2, 16, True, False)
            _radix_pass(kB, vB, kA, vA, c3, 24, False, False)

        r0 = wid * rows_per
        pltpu.async_copy(x_hbm.at[r0], rawA, semA)

        def row_body(rloc, carry):
            r = r0 + rloc
            pltpu.async_copy(y_hbm.at[r], rawB, semB)
            pltpu.make_async_copy(x_hbm.at[r], rawA, semA).wait()
            sort_array(rawA)
            axx = _rank_scatter(kA, vA, st, rx)
            rn = jnp.minimum(r + 1, b - 1)
            pltpu.async_copy(x_hbm.at[rn], rawA, semA)
            pltpu.make_async_copy(y_hbm.at[r], rawB, semB).wait()
            sort_array(rawB)
            axy, ayy = _rank_dot(kA, vA, st, rx)
            num = axy
            den2 = axx * ayy
            idx_n = jnp.full((16,), rloc, jnp.int32)
            idx_d = jnp.full((16,), rloc + 8, jnp.int32)
            lane0 = _iota16() == 0
            plsc.store_scatter(res, [idx_n], jnp.full((16,), num), mask=lane0)
            plsc.store_scatter(res, [idx_d], jnp.full((16,), den2), mask=lane0)
            return carry

        lax.fori_loop(0, rows_per, row_body, 0)
        pltpu.make_async_copy(x_hbm.at[r0], rawA, semA).wait()
        pltpu.sync_copy(res, out_hbm.at[wid])

    k = pl.kernel(
        body,
        out_type=jax.ShapeDtypeStruct((nworkers, 16), jnp.float32),
        mesh=mesh,
        compiler_params=pltpu.CompilerParams(needs_layout_passes=False),
        scratch_types=[
            pltpu.VMEM((_N,), jnp.float32),
            pltpu.VMEM((_N,), jnp.float32),
            pltpu.VMEM((_N,), jnp.int32),
            pltpu.VMEM((_N,), jnp.int32),
            pltpu.VMEM((_N,), jnp.int32),
            pltpu.VMEM((_N,), jnp.int32),
            pltpu.VMEM((_N,), jnp.int32),
            pltpu.VMEM((_N,), jnp.int32),
            pltpu.VMEM((_N,), jnp.int32),
            pltpu.VMEM((_N,), jnp.int32),
            pltpu.VMEM((_N,), jnp.int32),
            pltpu.VMEM((_N,), jnp.float32),
            pltpu.VMEM((16,), jnp.float32),
            pltpu.SemaphoreType.DMA,
            pltpu.SemaphoreType.DMA,
        ],
    )
    out = k(pred_y, true_y)
    num = out[:, 0:8].reshape(b)
    den2 = out[:, 8:16].reshape(b)
    return num / jnp.sqrt(den2 + _EPS)


# speedup vs baseline: 1.8754x; 1.0994x over previous
import jax
import jax.numpy as jnp
from jax import lax
from jax.experimental import pallas as pl
from jax.experimental.pallas import tpu as pltpu
from jax.experimental.pallas import tpu_sc as plsc

_N = 4096
_NV = _N // 16
_EPS = 1e-8
_BIG = _N
_MININT = -2147483648
_UNROLL = 8


def _iota16():
    return lax.iota(jnp.int32, 16)


def _keys_from_raw(x):
    x = jnp.where(x == 0.0, 0.0, x)
    i = lax.bitcast_convert_type(x, jnp.int32)
    return jnp.where(i < 0, ~i, i | jnp.int32(_MININT))


def _pass_pair(skx, svx, dkx, dvx, cx, sky, svy, dky, dvy, cy,
               shift, twist_out, first, rawx=None, rawy=None):
    ones = jnp.ones((16,), jnp.int32)

    def digits(k):
        d = jnp.bitwise_and(lax.shift_right_logical(k, shift), 255)
        return (d << 4) + _iota16()

    def s1(b, c):
        off = b * 16
        if rawx is not None:
            kx = _keys_from_raw(rawx[pl.ds(off, 16)])
            skx[pl.ds(off, 16)] = kx
            ky = _keys_from_raw(rawy[pl.ds(off, 16)])
            sky[pl.ds(off, 16)] = ky
        else:
            kx = skx[pl.ds(off, 16)]
            ky = sky[pl.ds(off, 16)]
        plsc.addupdate_scatter(cx, [digits(kx)], ones)
        plsc.addupdate_scatter(cy, [digits(ky)], ones)
        return c

    lax.fori_loop(0, _NV, s1, 0, unroll=_UNROLL)

    def csum(dg, carry):
        carx, cary = carry
        c0x = cx[pl.ds(dg * 16, 16)]
        inclx = plsc.cumsum(c0x)
        totx = lax.reduce_sum(c0x, (0,))
        cx[pl.ds(dg * 16, 16)] = inclx - c0x + carx
        c0y = cy[pl.ds(dg * 16, 16)]
        incly = plsc.cumsum(c0y)
        toty = lax.reduce_sum(c0y, (0,))
        cy[pl.ds(dg * 16, 16)] = incly - c0y + cary
        return carx + totx, cary + toty

    lax.fori_loop(0, _NV, csum, (jnp.int32(0), jnp.int32(0)), unroll=_UNROLL)

    def twist(pos):
        if twist_out:
            return (jnp.bitwise_and(pos, 255) << 4) | lax.shift_right_logical(
                pos, 8
            )
        return pos

    def s2(b, c):
        off = b * 16
        kx = skx[pl.ds(off, 16)]
        idxx = digits(kx)
        posx = plsc.load_gather(cx, [idxx])
        ky = sky[pl.ds(off, 16)]
        idxy = digits(ky)
        posy = plsc.load_gather(cy, [idxy])
        vx = _iota16() + off if first else svx[pl.ds(off, 16)]
        vy = _iota16() + off if first else svy[pl.ds(off, 16)]
        wx = twist(posx)
        wy = twist(posy)
        plsc.store_scatter(dkx, [wx], kx)
        plsc.store_scatter(dvx, [wx], vx)
        plsc.addupdate_scatter(cx, [idxx], ones)
        plsc.store_scatter(dky, [wy], ky)
        plsc.store_scatter(dvy, [wy], vy)
        plsc.addupdate_scatter(cy, [idxy], ones)
        return c

    lax.fori_loop(0, _NV, s2, 0, unroll=_UNROLL)


def _rank_pair(kfx, vfx, stx, rx, kfy, vfy, sty, ry):
    def fwd(b, carry):
        cax, cay = carry
        off = b * 16
        pidx = _iota16() + off
        pm1 = jnp.maximum(pidx - 1, 0)
        kx = kfx[pl.ds(off, 16)]
        prevx = plsc.load_gather(kfx, [pm1])
        bndx = jnp.logical_or(kx != prevx, pidx == 0)
        cmx = jnp.maximum(plsc.cummax(jnp.where(bndx, pidx, 0)), cax)
        stx[pl.ds(off, 16)] = cmx
        ky = kfy[pl.ds(off, 16)]
        prevy = plsc.load_gather(kfy, [pm1])
        bndy = jnp.logical_or(ky != prevy, pidx == 0)
        cmy = jnp.maximum(plsc.cummax(jnp.where(bndy, pidx, 0)), cay)
        sty[pl.ds(off, 16)] = cmy
        return lax.reduce_max(cmx, (0,)), lax.reduce_max(cmy, (0,))

    lax.fori_loop(0, _NV, fwd, (jnp.int32(0), jnp.int32(0)), unroll=_UNROLL)

    def bwd(t, carry):
        ecx, ecy, axx, ayy = carry
        b = _NV - 1 - t
        off = b * 16
        pidx = _iota16() + off
        pp1 = jnp.minimum(pidx + 1, _N - 1)
        last = pidx == _N - 1
        cshift = 1.0 - (_N + 1) / 2.0

        kx = kfx[pl.ds(off, 16)]
        nxtx = plsc.load_gather(kfx, [pp1])
        endbx = jnp.logical_or(kx != nxtx, last)
        candx = jnp.where(endbx, pidx, _BIG)
        sfxx = lax.rev(-plsc.cummax(-lax.rev(candx, (0,))), (0,))
        endx = jnp.minimum(sfxx, ecx)
        sx = stx[pl.ds(off, 16)]
        rcx = (sx + endx).astype(jnp.float32) * 0.5 + cshift
        plsc.store_scatter(rx, [vfx[pl.ds(off, 16)]], rcx)

        ky = kfy[pl.ds(off, 16)]
        nxty = plsc.load_gather(kfy, [pp1])
        endby = jnp.logical_or(ky != nxty, last)
        candy = jnp.where(endby, pidx, _BIG)
        sfxy = lax.rev(-plsc.cummax(-lax.rev(candy, (0,))), (0,))
        endy = jnp.minimum(sfxy, ecy)
        sy = sty[pl.ds(off, 16)]
        rcy = (sy + endy).astype(jnp.float32) * 0.5 + cshift
        plsc.store_scatter(ry, [vfy[pl.ds(off, 16)]], rcy)

        return (
            lax.reduce_min(endx, (0,)),
            lax.reduce_min(endy, (0,)),
            axx + rcx * rcx,
            ayy + rcy * rcy,
        )

    z = jnp.zeros((16,), jnp.float32)
    _, _, axx, ayy = lax.fori_loop(
        0, _NV, bwd, (jnp.int32(_N), jnp.int32(_N), z, z), unroll=_UNROLL
    )
    return lax.reduce_sum(axx, (0,)), lax.reduce_sum(ayy, (0,))


def kernel(pred_y, true_y):
    b, n = pred_y.shape
    mesh = plsc.VectorSubcoreMesh(core_axis_name="c", subcore_axis_name="s")
    nworkers = mesh.num_cores * mesh.num_subcores
    rows_per = b // nworkers

    def body(x_hbm, y_hbm, out_hbm, rawA, rawB,
             kAx, kBx, vAx, vBx, kAy, kBy, vAy, vBy,
             c0x, c1x, c2x, c3x, c0y, c1y, c2y, c3y,
             stx, sty, rx, ry, res, semA, semB):
        wid = lax.axis_index("s") * mesh.num_cores + lax.axis_index("c")
        zeros = jnp.zeros((16,), jnp.int32)
        r0 = wid * rows_per
        pltpu.async_copy(x_hbm.at[r0], rawA, semA)
        pltpu.async_copy(y_hbm.at[r0], rawB, semB)

        def row_body(rloc, carry):
            r = r0 + rloc
            pltpu.make_async_copy(x_hbm.at[r], rawA, semA).wait()
            pltpu.make_async_copy(y_hbm.at[r], rawB, semB).wait()

            def zero(i, c):
                off = i * 16
                c0x[pl.ds(off, 16)] = zeros
                c1x[pl.ds(off, 16)] = zeros
                c2x[pl.ds(off, 16)] = zeros
                c3x[pl.ds(off, 16)] = zeros
                c0y[pl.ds(off, 16)] = zeros
                c1y[pl.ds(off, 16)] = zeros
                c2y[pl.ds(off, 16)] = zeros
                c3y[pl.ds(off, 16)] = zeros
                return c

            lax.fori_loop(0, _NV, zero, 0, unroll=_UNROLL)
            _pass_pair(kAx, vAx, kBx, vBx, c0x, kAy, vAy, kBy, vBy, c0y,
                       0, True, True, rawx=rawA, rawy=rawB)
            rn = jnp.minimum(r + 1, b - 1)
            pltpu.async_copy(x_hbm.at[rn], rawA, semA)
            pltpu.async_copy(y_hbm.at[rn], rawB, semB)
            _pass_pair(kBx, vBx, kAx, vAx, c1x, kBy, vBy, kAy, vAy, c1y,
                       8, True, False)
            _pass_pair(kAx, vAx, kBx, vBx, c2x, kAy, vAy, kBy, vBy, c2y,
                       16, True, False)
            _pass_pair(kBx, vBx, kAx, vAx, c3x, kBy, vBy, kAy, vAy, c3y,
                       24, False, False)
            axx, ayy = _rank_pair(kAx, vAx, stx, rx, kAy, vAy, sty, ry)

            def dot_body(i, c):
                off = i * 16
                return c + rx[pl.ds(off, 16)] * ry[pl.ds(off, 16)]

            z = jnp.zeros((16,), jnp.float32)
            axy = lax.fori_loop(0, _NV, dot_body, z, unroll=_UNROLL)
            num = lax.reduce_sum(axy, (0,))
            den2 = axx * ayy
            idx_n = jnp.full((16,), rloc, jnp.int32)
            idx_d = jnp.full((16,), rloc + 8, jnp.int32)
            lane0 = _iota16() == 0
            plsc.store_scatter(res, [idx_n], jnp.full((16,), num), mask=lane0)
            plsc.store_scatter(res, [idx_d], jnp.full((16,), den2), mask=lane0)
            return carry

        lax.fori_loop(0, rows_per, row_body, 0)
        pltpu.make_async_copy(x_hbm.at[r0], rawA, semA).wait()
        pltpu.make_async_copy(y_hbm.at[r0], rawB, semB).wait()
        pltpu.sync_copy(res, out_hbm.at[wid])

    vm_i = pltpu.VMEM((_N,), jnp.int32)
    vm_f = pltpu.VMEM((_N,), jnp.float32)
    k = pl.kernel(
        body,
        out_type=jax.ShapeDtypeStruct((nworkers, 16), jnp.float32),
        mesh=mesh,
        compiler_params=pltpu.CompilerParams(needs_layout_passes=False),
        scratch_types=[
            vm_f, vm_f,
            vm_i, vm_i, vm_i, vm_i,
            vm_i, vm_i, vm_i, vm_i,
            vm_i, vm_i, vm_i, vm_i,
            vm_i, vm_i, vm_i, vm_i,
            vm_i, vm_i,
            vm_f, vm_f,
            pltpu.VMEM((16,), jnp.float32),
            pltpu.SemaphoreType.DMA,
            pltpu.SemaphoreType.DMA,
        ],
    )
    out = k(pred_y, true_y)
    num = out[:, 0:8].reshape(b)
    den2 = out[:, 8:16].reshape(b)
    return num / jnp.sqrt(den2 + _EPS)


# speedup vs baseline: 1.8756x; 1.0001x over previous
import jax
import jax.numpy as jnp
from jax import lax
from jax.experimental import pallas as pl
from jax.experimental.pallas import tpu as pltpu
from jax.experimental.pallas import tpu_sc as plsc

_N = 4096
_NV = _N // 16
_EPS = 1e-8
_BIG = _N
_MININT = -2147483648
_UNROLL = 8


def _iota16():
    return lax.iota(jnp.int32, 16)


def _keys_from_raw(x):
    x = jnp.where(x == 0.0, 0.0, x)
    i = lax.bitcast_convert_type(x, jnp.int32)
    return jnp.where(i < 0, ~i, i | jnp.int32(_MININT))


def _pass_pair(skx, svx, dkx, dvx, cx, sky, svy, dky, dvy, cy,
               shift, twist_out, first, rawx=None, rawy=None):
    ones = jnp.ones((16,), jnp.int32)

    def digits(k):
        d = jnp.bitwise_and(lax.shift_right_logical(k, shift), 255)
        return (d << 4) + _iota16()

    def s1(b, c):
        off = b * 16
        if rawx is not None:
            kx = _keys_from_raw(rawx[pl.ds(off, 16)])
            skx[pl.ds(off, 16)] = kx
            ky = _keys_from_raw(rawy[pl.ds(off, 16)])
            sky[pl.ds(off, 16)] = ky
        else:
            kx = skx[pl.ds(off, 16)]
            ky = sky[pl.ds(off, 16)]
        plsc.addupdate_scatter(cx, [digits(kx)], ones)
        plsc.addupdate_scatter(cy, [digits(ky)], ones)
        return c

    lax.fori_loop(0, _NV, s1, 0, unroll=_UNROLL)

    def csum(dg, carry):
        carx, cary = carry
        c0x = cx[pl.ds(dg * 16, 16)]
        inclx = plsc.cumsum(c0x)
        cx[pl.ds(dg * 16, 16)] = inclx - c0x + carx
        c0y = cy[pl.ds(dg * 16, 16)]
        incly = plsc.cumsum(c0y)
        cy[pl.ds(dg * 16, 16)] = incly - c0y + cary
        return carx + inclx[15], cary + incly[15]

    lax.fori_loop(0, _NV, csum, (jnp.int32(0), jnp.int32(0)), unroll=_UNROLL)

    def twist(pos):
        if twist_out:
            return (jnp.bitwise_and(pos, 255) << 4) | lax.shift_right_logical(
                pos, 8
            )
        return pos

    def s2(b, c):
        off = b * 16
        kx = skx[pl.ds(off, 16)]
        idxx = digits(kx)
        posx = plsc.load_gather(cx, [idxx])
        ky = sky[pl.ds(off, 16)]
        idxy = digits(ky)
        posy = plsc.load_gather(cy, [idxy])
        vx = _iota16() + off if first else svx[pl.ds(off, 16)]
        vy = _iota16() + off if first else svy[pl.ds(off, 16)]
        wx = twist(posx)
        wy = twist(posy)
        plsc.store_scatter(dkx, [wx], kx)
        plsc.store_scatter(dvx, [wx], vx)
        plsc.addupdate_scatter(cx, [idxx], ones)
        plsc.store_scatter(dky, [wy], ky)
        plsc.store_scatter(dvy, [wy], vy)
        plsc.addupdate_scatter(cy, [idxy], ones)
        return c

    lax.fori_loop(0, _NV, s2, 0, unroll=_UNROLL)


def _rank_pair(kfx, vfx, stx, rx, kfy, vfy, sty, ry):
    def fwd(b, carry):
        cax, cay = carry
        off = b * 16
        pidx = _iota16() + off
        pm1 = jnp.maximum(pidx - 1, 0)
        kx = kfx[pl.ds(off, 16)]
        prevx = plsc.load_gather(kfx, [pm1])
        bndx = jnp.logical_or(kx != prevx, pidx == 0)
        cmx = jnp.maximum(plsc.cummax(jnp.where(bndx, pidx, 0)), cax)
        stx[pl.ds(off, 16)] = cmx
        ky = kfy[pl.ds(off, 16)]
        prevy = plsc.load_gather(kfy, [pm1])
        bndy = jnp.logical_or(ky != prevy, pidx == 0)
        cmy = jnp.maximum(plsc.cummax(jnp.where(bndy, pidx, 0)), cay)
        sty[pl.ds(off, 16)] = cmy
        return cmx[15], cmy[15]

    lax.fori_loop(0, _NV, fwd, (jnp.int32(0), jnp.int32(0)), unroll=_UNROLL)

    def bwd(t, carry):
        ecx, ecy, axx, ayy = carry
        b = _NV - 1 - t
        off = b * 16
        pidx = _iota16() + off
        pp1 = jnp.minimum(pidx + 1, _N - 1)
        last = pidx == _N - 1
        cshift = 1.0 - (_N + 1) / 2.0

        kx = kfx[pl.ds(off, 16)]
        nxtx = plsc.load_gather(kfx, [pp1])
        endbx = jnp.logical_or(kx != nxtx, last)
        candx = jnp.where(endbx, pidx, _BIG)
        sfxx = lax.rev(-plsc.cummax(-lax.rev(candx, (0,))), (0,))
        endx = jnp.minimum(sfxx, ecx)
        sx = stx[pl.ds(off, 16)]
        rcx = (sx + endx).astype(jnp.float32) * 0.5 + cshift
        plsc.store_scatter(rx, [vfx[pl.ds(off, 16)]], rcx)

        ky = kfy[pl.ds(off, 16)]
        nxty = plsc.load_gather(kfy, [pp1])
        endby = jnp.logical_or(ky != nxty, last)
        candy = jnp.where(endby, pidx, _BIG)
        sfxy = lax.rev(-plsc.cummax(-lax.rev(candy, (0,))), (0,))
        endy = jnp.minimum(sfxy, ecy)
        sy = sty[pl.ds(off, 16)]
        rcy = (sy + endy).astype(jnp.float32) * 0.5 + cshift
        plsc.store_scatter(ry, [vfy[pl.ds(off, 16)]], rcy)

        return endx[0], endy[0], axx + rcx * rcx, ayy + rcy * rcy

    z = jnp.zeros((16,), jnp.float32)
    _, _, axx, ayy = lax.fori_loop(
        0, _NV, bwd, (jnp.int32(_N), jnp.int32(_N), z, z), unroll=_UNROLL
    )
    return lax.reduce_sum(axx, (0,)), lax.reduce_sum(ayy, (0,))


def kernel(pred_y, true_y):
    b, n = pred_y.shape
    mesh = plsc.VectorSubcoreMesh(core_axis_name="c", subcore_axis_name="s")
    nworkers = mesh.num_cores * mesh.num_subcores
    rows_per = b // nworkers

    def body(x_hbm, y_hbm, out_hbm, rawA, rawB,
             kAx, kBx, vAx, vBx, kAy, kBy, vAy, vBy,
             c0x, c1x, c2x, c3x, c0y, c1y, c2y, c3y,
             stx, sty, rx, ry, res, semA, semB):
        wid = lax.axis_index("s") * mesh.num_cores + lax.axis_index("c")
        zeros = jnp.zeros((16,), jnp.int32)
        r0 = wid * rows_per
        pltpu.async_copy(x_hbm.at[r0], rawA, semA)
        pltpu.async_copy(y_hbm.at[r0], rawB, semB)

        def row_body(rloc, carry):
            r = r0 + rloc
            pltpu.make_async_copy(x_hbm.at[r], rawA, semA).wait()
            pltpu.make_async_copy(y_hbm.at[r], rawB, semB).wait()

            def zero(i, c):
                off = i * 16
                c0x[pl.ds(off, 16)] = zeros
                c1x[pl.ds(off, 16)] = zeros
                c2x[pl.ds(off, 16)] = zeros
                c3x[pl.ds(off, 16)] = zeros
                c0y[pl.ds(off, 16)] = zeros
                c1y[pl.ds(off, 16)] = zeros
                c2y[pl.ds(off, 16)] = zeros
                c3y[pl.ds(off, 16)] = zeros
                return c

            lax.fori_loop(0, _NV, zero, 0, unroll=_UNROLL)
            _pass_pair(kAx, vAx, kBx, vBx, c0x, kAy, vAy, kBy, vBy, c0y,
                       0, True, True, rawx=rawA, rawy=rawB)
            rn = jnp.minimum(r + 1, b - 1)
            pltpu.async_copy(x_hbm.at[rn], rawA, semA)
            pltpu.async_copy(y_hbm.at[rn], rawB, semB)
            _pass_pair(kBx, vBx, kAx, vAx, c1x, kBy, vBy, kAy, vAy, c1y,
                       8, True, False)
            _pass_pair(kAx, vAx, kBx, vBx, c2x, kAy, vAy, kBy, vBy, c2y,
                       16, True, False)
            _pass_pair(kBx, vBx, kAx, vAx, c3x, kBy, vBy, kAy, vAy, c3y,
                       24, False, False)
            axx, ayy = _rank_pair(kAx, vAx, stx, rx, kAy, vAy, sty, ry)

            def dot_body(i, c):
                off = i * 16
                return c + rx[pl.ds(off, 16)] * ry[pl.ds(off, 16)]

            z = jnp.zeros((16,), jnp.float32)
            axy = lax.fori_loop(0, _NV, dot_body, z, unroll=_UNROLL)
            num = lax.reduce_sum(axy, (0,))
            den2 = axx * ayy
            idx_n = jnp.full((16,), rloc, jnp.int32)
            idx_d = jnp.full((16,), rloc + 8, jnp.int32)
            lane0 = _iota16() == 0
            plsc.store_scatter(res, [idx_n], jnp.full((16,), num), mask=lane0)
            plsc.store_scatter(res, [idx_d], jnp.full((16,), den2), mask=lane0)
            return carry

        lax.fori_loop(0, rows_per, row_body, 0)
        pltpu.make_async_copy(x_hbm.at[r0], rawA, semA).wait()
        pltpu.make_async_copy(y_hbm.at[r0], rawB, semB).wait()
        pltpu.sync_copy(res, out_hbm.at[wid])

    vm_i = pltpu.VMEM((_N,), jnp.int32)
    vm_f = pltpu.VMEM((_N,), jnp.float32)
    k = pl.kernel(
        body,
        out_type=jax.ShapeDtypeStruct((nworkers, 16), jnp.float32),
        mesh=mesh,
        compiler_params=pltpu.CompilerParams(needs_layout_passes=False),
        scratch_types=[
            vm_f, vm_f,
            vm_i, vm_i, vm_i, vm_i,
            vm_i, vm_i, vm_i, vm_i,
            vm_i, vm_i, vm_i, vm_i,
            vm_i, vm_i, vm_i, vm_i,
            vm_i, vm_i,
            vm_f, vm_f,
            pltpu.VMEM((16,), jnp.float32),
            pltpu.SemaphoreType.DMA,
            pltpu.SemaphoreType.DMA,
        ],
    )
    out = k(pred_y, true_y)
    num = out[:, 0:8].reshape(b)
    den2 = out[:, 8:16].reshape(b)
    return num / jnp.sqrt(den2 + _EPS)
